# baseline (speedup 1.0000x reference)
import functools

import jax
import jax.numpy as jnp
from jax import lax
from jax.experimental import pallas as pl
from jax.experimental.pallas import tpu as pltpu
from jax.experimental.pallas import tpu_sc as plsc

_NC = 2
_NS = 16
_NW = _NC * _NS

_BATCH = 4
_SEQ = 2048
_D = 768
_B = _BATCH * _SEQ
_B_PER_W = _B // _NW
_W_PER_ROW = _SEQ // _B_PER_W
_CH = 16
_NCH = _B_PER_W // _CH
_NBUF = 8


def _embed_lookup(tokens, w_e):
    mesh = plsc.VectorSubcoreMesh(core_axis_name="c", subcore_axis_name="s")

    @functools.partial(
        pl.kernel,
        mesh=mesh,
        out_type=jax.ShapeDtypeStruct((_BATCH, _SEQ, _D), jnp.float32),
        scratch_types=(
            [pltpu.VMEM((_B_PER_W,), jnp.int32)]
            + [pltpu.VMEM((_CH, _D), jnp.float32) for _ in range(_NBUF)]
            + [pltpu.SemaphoreType.DMA for _ in range(2 * _NBUF)]
        ),
    )
    def body(tok_hbm, table_hbm, out_hbm, idx_v, *rest):
        bufs = rest[:_NBUF]
        gsems = rest[_NBUF:2 * _NBUF]
        osems = rest[2 * _NBUF:]
        wid = lax.axis_index("s") * _NC + lax.axis_index("c")
        row = wid // _W_PER_ROW
        col0 = (wid % _W_PER_ROW) * _B_PER_W
        pltpu.sync_copy(tok_hbm.at[row, pl.ds(col0, _B_PER_W)], idx_v)

        gh = [None] * _NBUF
        oh = [None] * _NBUF
        for b in range(_NBUF):
            gh[b] = pltpu.async_copy(
                table_hbm.at[idx_v.at[pl.ds(b * _CH, _CH)]], bufs[b], gsems[b])
        for c in range(_NCH):
            b = c % _NBUF
            if c >= 1 and c - 1 + _NBUF < _NCH:
                pb = (c - 1) % _NBUF
                oh[pb].wait()
                gh[pb] = pltpu.async_copy(
                    table_hbm.at[idx_v.at[pl.ds((c - 1 + _NBUF) * _CH, _CH)]],
                    bufs[pb], gsems[pb])
            gh[b].wait()
            oh[b] = pltpu.async_copy(
                bufs[b], out_hbm.at[row, pl.ds(col0 + c * _CH, _CH)], osems[b])
        for c in range(max(0, _NCH - _NBUF), _NCH):
            oh[c % _NBUF].wait()

    return body(tokens, w_e)


def kernel(tokens, W_E):
    return _embed_lookup(tokens, W_E)


# speedup vs baseline: 1.0229x; 1.0229x over previous
import functools

import jax
import jax.numpy as jnp
from jax import lax
from jax.experimental import pallas as pl
from jax.experimental.pallas import tpu as pltpu
from jax.experimental.pallas import tpu_sc as plsc

_NC = 2
_NS = 16
_NW = _NC * _NS

_BATCH = 4
_SEQ = 2048
_D = 768
_B = _BATCH * _SEQ
_B_PER_W = _B // _NW
_W_PER_ROW = _SEQ // _B_PER_W
_CH = 32
_NCH = _B_PER_W // _CH
_NBUF = 5


def _embed_lookup(tokens, w_e):
    mesh = plsc.VectorSubcoreMesh(core_axis_name="c", subcore_axis_name="s")

    @functools.partial(
        pl.kernel,
        mesh=mesh,
        out_type=jax.ShapeDtypeStruct((_BATCH, _SEQ, _D), jnp.float32),
        scratch_types=(
            [pltpu.VMEM((_B_PER_W,), jnp.int32)]
            + [pltpu.VMEM((_CH, _D), jnp.float32) for _ in range(_NBUF)]
            + [pltpu.SemaphoreType.DMA for _ in range(2 * _NBUF)]
        ),
    )
    def body(tok_hbm, table_hbm, out_hbm, idx_v, *rest):
        bufs = rest[:_NBUF]
        gsems = rest[_NBUF:2 * _NBUF]
        osems = rest[2 * _NBUF:]
        wid = lax.axis_index("s") * _NC + lax.axis_index("c")
        row = wid // _W_PER_ROW
        col0 = (wid % _W_PER_ROW) * _B_PER_W
        pltpu.sync_copy(tok_hbm.at[row, pl.ds(col0, _B_PER_W)], idx_v)

        gh = [None] * _NBUF
        oh = [None] * _NBUF
        for b in range(_NBUF):
            gh[b] = pltpu.async_copy(
                table_hbm.at[idx_v.at[pl.ds(b * _CH, _CH)]], bufs[b], gsems[b])
        for c in range(_NCH):
            b = c % _NBUF
            if c >= 1 and c - 1 + _NBUF < _NCH:
                pb = (c - 1) % _NBUF
                oh[pb].wait()
                gh[pb] = pltpu.async_copy(
                    table_hbm.at[idx_v.at[pl.ds((c - 1 + _NBUF) * _CH, _CH)]],
                    bufs[pb], gsems[pb])
            gh[b].wait()
            oh[b] = pltpu.async_copy(
                bufs[b], out_hbm.at[row, pl.ds(col0 + c * _CH, _CH)], osems[b])
        for c in range(max(0, _NCH - _NBUF), _NCH):
            oh[c % _NBUF].wait()

    return body(tokens, w_e)


def kernel(tokens, W_E):
    return _embed_lookup(tokens, W_E)
